# baseline (speedup 1.0000x reference)
import functools

import jax
import jax.numpy as jnp
from jax import lax
from jax.experimental import pallas as pl
from jax.experimental.pallas import tpu as pltpu
from jax.experimental.pallas import tpu_sc as plsc

_T_MAX = 1000
_BETA_MIN = 0.0001
_BETA_MAX = 0.02


@functools.partial(jax.jit, static_argnums=())
def kernel(t, abars):
    B = t.shape[0]
    Tm = abars.shape[0]
    L = 16
    mesh = plsc.VectorSubcoreMesh(core_axis_name="c", subcore_axis_name="s",
                                  num_cores=1)
    NC, NS = mesh.num_cores, mesh.num_subcores
    NW = NC * NS
    b_per_w = B // NW
    nreg = b_per_w // L
    scale = jnp.float32((_BETA_MAX - _BETA_MIN) / _T_MAX)

    @functools.partial(
        pl.kernel,
        mesh=mesh,
        compiler_params=pltpu.CompilerParams(
            needs_layout_passes=False,
            disable_bounds_checks=True,
            disable_semaphore_checks=True,
            skip_device_barrier=True,
        ),
        out_type=(
            jax.ShapeDtypeStruct((B,), jnp.float32),
            jax.ShapeDtypeStruct((B,), jnp.float32),
        ),
        scratch_types=[
            pltpu.VMEM((Tm,), jnp.float32),
            pltpu.VMEM((b_per_w,), jnp.int32),
            pltpu.VMEM((b_per_w,), jnp.float32),
            pltpu.VMEM((b_per_w,), jnp.float32),
            pltpu.SemaphoreType.DMA,
            pltpu.SemaphoreType.DMA,
        ],
    )
    def run(t_hbm, abars_hbm, at_hbm, betas_hbm,
            abars_v, t_v, at_v, betas_v, sem_a, sem_t):
        wid = lax.axis_index("s") * NC + lax.axis_index("c")
        base = wid * b_per_w
        cp_a = pltpu.async_copy(abars_hbm, abars_v, sem_a)
        cp_t = pltpu.async_copy(t_hbm.at[pl.ds(base, b_per_w)], t_v, sem_t)
        cp_t.wait()
        for i in range(nreg):
            sl = pl.ds(i * L, L)
            betas_v[sl] = _BETA_MIN + t_v[sl].astype(jnp.float32) * scale
        cp_out_b = pltpu.async_copy(betas_v, betas_hbm.at[pl.ds(base, b_per_w)], sem_t)
        cp_a.wait()
        for i in range(nreg):
            sl = pl.ds(i * L, L)
            at_v[sl] = plsc.load_gather(abars_v, [t_v[sl]])
        cp_out_a = pltpu.async_copy(at_v, at_hbm.at[pl.ds(base, b_per_w)], sem_a)
        cp_out_b.wait()
        cp_out_a.wait()

    at, betas = run(t, abars)
    return (at, betas)


# speedup vs baseline: 1.0182x; 1.0182x over previous
import functools

import jax
import jax.numpy as jnp
from jax import lax
from jax.experimental import pallas as pl
from jax.experimental.pallas import tpu as pltpu
from jax.experimental.pallas import tpu_sc as plsc

_T_MAX = 1000
_BETA_MIN = 0.0001
_BETA_MAX = 0.02


@functools.partial(jax.jit, static_argnums=())
def kernel(t, abars):
    B = t.shape[0]
    Tm = abars.shape[0]
    L = 16
    mesh = plsc.VectorSubcoreMesh(core_axis_name="c", subcore_axis_name="s",
                                  num_cores=1)
    NC, NS = mesh.num_cores, mesh.num_subcores
    NW = NC * NS
    b_per_w = B // NW
    nreg = b_per_w // L
    scale = jnp.float32((_BETA_MAX - _BETA_MIN) / _T_MAX)

    @functools.partial(
        pl.kernel,
        mesh=mesh,
        compiler_params=pltpu.CompilerParams(
            needs_layout_passes=False,
            disable_bounds_checks=True,
            disable_semaphore_checks=True,
            skip_device_barrier=True,
        ),
        out_type=(
            jax.ShapeDtypeStruct((B,), jnp.float32),
            jax.ShapeDtypeStruct((B,), jnp.float32),
        ),
        scratch_types=[
            pltpu.VMEM((Tm,), jnp.float32),
            pltpu.VMEM((b_per_w,), jnp.int32),
            pltpu.VMEM((b_per_w,), jnp.float32),
            pltpu.VMEM((b_per_w,), jnp.float32),
            pltpu.SemaphoreType.DMA,
            pltpu.SemaphoreType.DMA,
        ],
    )
    def run(t_hbm, abars_hbm, at_hbm, betas_hbm,
            abars_v, t_v, at_v, betas_v, sem_a, sem_t):
        wid = lax.axis_index("s") * NC + lax.axis_index("c")
        base = wid * b_per_w
        cp_a = pltpu.async_copy(abars_hbm, abars_v, sem_a)
        cp_t = pltpu.async_copy(t_hbm.at[pl.ds(base, b_per_w)], t_v, sem_t)
        cp_t.wait()

        def betas_body(i, carry):
            sl = pl.ds(i * L, L)
            betas_v[sl] = _BETA_MIN + t_v[sl].astype(jnp.float32) * scale
            return carry

        lax.fori_loop(0, nreg, betas_body, 0)
        cp_out_b = pltpu.async_copy(betas_v, betas_hbm.at[pl.ds(base, b_per_w)], sem_t)
        cp_a.wait()

        def gather_body(i, carry):
            sl = pl.ds(i * L, L)
            at_v[sl] = plsc.load_gather(abars_v, [t_v[sl]])
            return carry

        lax.fori_loop(0, nreg, gather_body, 0)
        cp_out_a = pltpu.async_copy(at_v, at_hbm.at[pl.ds(base, b_per_w)], sem_a)
        cp_out_b.wait()
        cp_out_a.wait()

    at, betas = run(t, abars)
    return (at, betas)


# speedup vs baseline: 1.0246x; 1.0063x over previous
import functools

import jax
import jax.numpy as jnp
from jax import lax
from jax.experimental import pallas as pl
from jax.experimental.pallas import tpu as pltpu
from jax.experimental.pallas import tpu_sc as plsc

_T_MAX = 1000
_BETA_MIN = 0.0001
_BETA_MAX = 0.02


@functools.partial(jax.jit, static_argnums=())
def kernel(t, abars):
    B = t.shape[0]
    Tm = abars.shape[0]
    L = 16
    mesh = plsc.VectorSubcoreMesh(core_axis_name="c", subcore_axis_name="s",
                                  num_cores=1)
    NC, NS = mesh.num_cores, mesh.num_subcores
    NW = NC * NS
    b_per_w = B // NW
    nreg = b_per_w // L
    scale = jnp.float32((_BETA_MAX - _BETA_MIN) / _T_MAX)

    @functools.partial(
        pl.kernel,
        mesh=mesh,
        compiler_params=pltpu.CompilerParams(
            needs_layout_passes=False,
            disable_bounds_checks=True,
            disable_semaphore_checks=True,
            skip_device_barrier=True,
        ),
        out_type=(
            jax.ShapeDtypeStruct((B,), jnp.float32),
            jax.ShapeDtypeStruct((B,), jnp.float32),
        ),
        scratch_types=[
            pltpu.VMEM((Tm,), jnp.float32),
            pltpu.VMEM((b_per_w,), jnp.int32),
            pltpu.VMEM((b_per_w,), jnp.float32),
            pltpu.VMEM((b_per_w,), jnp.float32),
            pltpu.SemaphoreType.DMA,
            pltpu.SemaphoreType.DMA,
        ],
    )
    def run(t_hbm, abars_hbm, at_hbm, betas_hbm,
            abars_v, t_v, at_v, betas_v, sem_a, sem_t):
        wid = lax.axis_index("s") * NC + lax.axis_index("c")
        base = wid * b_per_w
        cp_a = pltpu.async_copy(abars_hbm, abars_v, sem_a)
        cp_t = pltpu.async_copy(t_hbm.at[pl.ds(base, b_per_w)], t_v, sem_t)
        cp_t.wait()

        @plsc.parallel_loop(0, nreg, unroll=4)
        def betas_body(i):
            sl = pl.ds(i * L, L)
            betas_v[sl] = _BETA_MIN + t_v[sl].astype(jnp.float32) * scale

        cp_out_b = pltpu.async_copy(betas_v, betas_hbm.at[pl.ds(base, b_per_w)], sem_t)
        cp_a.wait()

        @plsc.parallel_loop(0, nreg, unroll=4)
        def gather_body(i):
            sl = pl.ds(i * L, L)
            at_v[sl] = plsc.load_gather(abars_v, [t_v[sl]])
        cp_out_a = pltpu.async_copy(at_v, at_hbm.at[pl.ds(base, b_per_w)], sem_a)
        cp_out_b.wait()
        cp_out_a.wait()

    at, betas = run(t, abars)
    return (at, betas)


# speedup vs baseline: 1.0310x; 1.0063x over previous
import functools

import jax
import jax.numpy as jnp
from jax import lax
from jax.experimental import pallas as pl
from jax.experimental.pallas import tpu as pltpu
from jax.experimental.pallas import tpu_sc as plsc

_T_MAX = 1000
_BETA_MIN = 0.0001
_BETA_MAX = 0.02


@functools.partial(jax.jit, static_argnums=())
def kernel(t, abars):
    B = t.shape[0]
    Tm = abars.shape[0]
    L = 16
    mesh = plsc.VectorSubcoreMesh(core_axis_name="c", subcore_axis_name="s",
                                  num_cores=1)
    NC, NS = mesh.num_cores, mesh.num_subcores
    NW = NC * NS
    b_per_w = B // NW
    nreg = b_per_w // L
    scale = jnp.float32((_BETA_MAX - _BETA_MIN) / _T_MAX)

    @functools.partial(
        pl.kernel,
        mesh=mesh,
        compiler_params=pltpu.CompilerParams(
            needs_layout_passes=False,
            disable_bounds_checks=True,
            disable_semaphore_checks=True,
            skip_device_barrier=True,
        ),
        out_type=(
            jax.ShapeDtypeStruct((B,), jnp.float32),
            jax.ShapeDtypeStruct((B,), jnp.float32),
        ),
        scratch_types=[
            pltpu.VMEM((Tm,), jnp.float32),
            pltpu.VMEM((b_per_w,), jnp.int32),
            pltpu.VMEM((b_per_w,), jnp.float32),
            pltpu.VMEM((b_per_w,), jnp.float32),
            pltpu.SemaphoreType.DMA,
            pltpu.SemaphoreType.DMA,
        ],
    )
    def run(t_hbm, abars_hbm, at_hbm, betas_hbm,
            abars_v, t_v, at_v, betas_v, sem_a, sem_t):
        wid = lax.axis_index("s") * NC + lax.axis_index("c")
        base = wid * b_per_w
        cp_a = pltpu.async_copy(abars_hbm, abars_v, sem_a)
        cp_t = pltpu.async_copy(t_hbm.at[pl.ds(base, b_per_w)], t_v, sem_t)
        cp_t.wait()
        cp_a.wait()

        @plsc.parallel_loop(0, nreg, unroll=4)
        def body(i):
            sl = pl.ds(i * L, L)
            tv = t_v[sl]
            at_v[sl] = plsc.load_gather(abars_v, [tv])
            betas_v[sl] = _BETA_MIN + tv.astype(jnp.float32) * scale

        cp_out_a = pltpu.async_copy(at_v, at_hbm.at[pl.ds(base, b_per_w)], sem_a)
        cp_out_b = pltpu.async_copy(betas_v, betas_hbm.at[pl.ds(base, b_per_w)], sem_t)
        cp_out_a.wait()
        cp_out_b.wait()

    at, betas = run(t, abars)
    return (at, betas)


# speedup vs baseline: 1.0352x; 1.0040x over previous
import functools

import jax
import jax.numpy as jnp
from jax import lax
from jax.experimental import pallas as pl
from jax.experimental.pallas import tpu as pltpu
from jax.experimental.pallas import tpu_sc as plsc

_T_MAX = 1000
_BETA_MIN = 0.0001
_BETA_MAX = 0.02


@functools.partial(jax.jit, static_argnums=())
def kernel(t, abars):
    B = t.shape[0]
    Tm = abars.shape[0]
    L = 16
    mesh = plsc.VectorSubcoreMesh(core_axis_name="c", subcore_axis_name="s",
                                  num_cores=1)
    NC, NS = mesh.num_cores, mesh.num_subcores
    NW = NC * NS
    b_per_w = B // NW
    nreg = b_per_w // L
    scale = jnp.float32((_BETA_MAX - _BETA_MIN) / _T_MAX)

    @functools.partial(
        pl.kernel,
        mesh=mesh,
        compiler_params=pltpu.CompilerParams(
            needs_layout_passes=False,
            disable_bounds_checks=True,
            disable_semaphore_checks=True,
            skip_device_barrier=True,
        ),
        out_type=(
            jax.ShapeDtypeStruct((B,), jnp.float32),
            jax.ShapeDtypeStruct((B,), jnp.float32),
        ),
        scratch_types=[
            pltpu.VMEM((Tm,), jnp.float32),
            pltpu.VMEM((b_per_w,), jnp.int32),
            pltpu.VMEM((b_per_w,), jnp.float32),
            pltpu.VMEM((b_per_w,), jnp.float32),
            pltpu.SemaphoreType.DMA,
            pltpu.SemaphoreType.DMA,
        ],
    )
    def run(t_hbm, abars_hbm, at_hbm, betas_hbm,
            abars_v, t_v, at_v, betas_v, sem_a, sem_t):
        wid = lax.axis_index("s") * NC + lax.axis_index("c")
        base = wid * b_per_w
        cp_a = pltpu.async_copy(abars_hbm, abars_v, sem_a)
        cp_t = pltpu.async_copy(t_hbm.at[pl.ds(base, b_per_w)], t_v, sem_t)
        cp_t.wait()
        cp_a.wait()

        @plsc.parallel_loop(0, nreg, unroll=8)
        def body(i):
            sl = pl.ds(i * L, L)
            tv = t_v[sl]
            at_v[sl] = plsc.load_gather(abars_v, [tv])
            betas_v[sl] = _BETA_MIN + tv.astype(jnp.float32) * scale

        cp_out_a = pltpu.async_copy(at_v, at_hbm.at[pl.ds(base, b_per_w)], sem_a)
        cp_out_b = pltpu.async_copy(betas_v, betas_hbm.at[pl.ds(base, b_per_w)], sem_t)
        cp_out_a.wait()
        cp_out_b.wait()

    at, betas = run(t, abars)
    return (at, betas)
